# baseline (speedup 1.0000x reference)
import functools
import math

import jax
import jax.numpy as jnp
from jax import lax
from jax.experimental import pallas as pl
from jax.experimental.pallas import tpu as pltpu
from jax.experimental.pallas import tpu_sc as plsc

_LOG_NORM_CONST = -0.5 * math.log(2 * math.pi)
_NUM_ITER = 3
_BETA = 0.9
_K = 512
_D = 64
_N = 2048


def _tree_last(x):
    n = x.shape[-1]
    while n > 1:
        h = n // 2
        x = x[..., :h] + x[..., h:n]
        n = h
    return x


def _red_k_sum(x):
    acc = x[0:8]
    for i in range(1, 64):
        acc = acc + x[i * 8:(i + 1) * 8]
    n = 8
    while n > 1:
        h = n // 2
        acc = acc[:h] + acc[h:n]
        n = h
    return acc


def _red_n_sum(x):
    acc = x[:, 0:128]
    for i in range(1, 16):
        acc = acc + x[:, i * 128:(i + 1) * 128]
    acc2 = acc[:, 0:8]
    for j in range(1, 16):
        acc2 = acc2 + acc[:, j * 8:(j + 1) * 8]
    return _tree_last(acc2)


def _ll_into(ll_ref, xf_ref, mu_ref, lv_ref):
    def chunk(c, _):
        base = pl.multiple_of(c * 8, 8)
        mu_c = mu_ref[pl.ds(base, 8)]
        lv_c = lv_ref[pl.ds(base, 8)]
        for nq in range(4):
            xq = xf_ref[nq * 512:(nq + 1) * 512, :]
            t = (-0.5 * (lv_c[:, None, :] + (xq[None, :, :] - mu_c[:, None, :]) ** 2
                         / jnp.exp(lv_c[:, None, :])) + _LOG_NORM_CONST)
            acc = None
            for c8 in range(8):
                part = _tree_last(t[..., c8 * 8:(c8 + 1) * 8])
                acc = part if acc is None else acc + part
            ll_ref[pl.ds(base, 8), nq * 512:(nq + 1) * 512] = acc[..., 0]
        return 0
    jax.lax.fori_loop(0, 64, chunk, 0)


_KB = 16


def _em_iter_body(xf_ref, mu_ref, lv_ref, munew_ref, lvnew_ref,
                  ll_ref, p_ref, den_ref):
    _ll_into(ll_ref, xf_ref, mu_ref, lv_ref)
    ll = ll_ref[:]
    m = jnp.max(ll, axis=0, keepdims=True)
    ex = jnp.exp(ll - m)
    s = _red_k_sum(ex)
    p = jnp.exp(ll - (m + jnp.log(s)))
    p_ref[:] = p
    n_k = _red_n_sum(p)
    den_ref[:] = n_k + 1e-6
    pb = p.astype(jnp.bfloat16)
    xb = xf_ref[:].astype(jnp.bfloat16)
    s1 = jax.lax.dot_general(pb, xb, (((1,), (0,)), ((), ())),
                             preferred_element_type=jnp.float32)
    munew_ref[:] = s1 / den_ref[:]

    xfv = xf_ref[:]

    def vblk(i, _):
        base = pl.multiple_of(i * _KB, 8)
        mu_b = munew_ref[pl.ds(base, _KB)]
        a2 = (xfv[None, :, :] - mu_b[:, None, :]) ** 2
        a2b = a2.astype(jnp.bfloat16)
        pb_b = p_ref[pl.ds(base, _KB)].astype(jnp.bfloat16)
        r = jax.lax.dot_general(pb_b[:, None, :], a2b,
                                (((2,), (1,)), ((0,), (0,))),
                                preferred_element_type=jnp.float32)
        var = r[:, 0, :] / den_ref[pl.ds(base, _KB)]
        lvnew_ref[pl.ds(base, _KB)] = jnp.log(jnp.maximum(var, 1e-6))
        return 0
    jax.lax.fori_loop(0, _K // _KB, vblk, 0)


def _em_iter(xf, mu, lv):
    return pl.pallas_call(
        _em_iter_body,
        out_shape=[
            jax.ShapeDtypeStruct((_K, _D), jnp.float32),
            jax.ShapeDtypeStruct((_K, _D), jnp.float32),
        ],
        scratch_shapes=[
            pltpu.VMEM((_K, _N), jnp.float32),
            pltpu.VMEM((_K, _N), jnp.float32),
            pltpu.VMEM((_K, 1), jnp.float32),
        ],
    )(xf, mu, lv)


def _final_body(xf_ref, mu_ref, lv_ref, emb_ref,
                enc_ref, embnew_ref, idx_ref, ll_ref):
    _ll_into(ll_ref, xf_ref, mu_ref, lv_ref)
    lik = jnp.exp(ll_ref[:])
    mx = jnp.max(lik, axis=0, keepdims=True)
    kio = lax.broadcasted_iota(jnp.int32, (_K, _N), 0)
    cand = jnp.where(lik == mx, kio, _K)
    idxr = jnp.min(cand, axis=0, keepdims=True)
    idx_col = jnp.transpose(idxr)
    nio = lax.broadcasted_iota(jnp.int32, (_N, _K), 1)
    enc_ref[:] = (nio == idx_col).astype(jnp.float32)
    idx_ref[:] = idx_col
    embnew_ref[:] = _BETA * emb_ref[:] + (1.0 - _BETA) * mu_ref[:]


def _final_k(xf, mu, lv, emb_mu):
    return pl.pallas_call(
        _final_body,
        out_shape=[
            jax.ShapeDtypeStruct((_N, _K), jnp.float32),
            jax.ShapeDtypeStruct((_K, _D), jnp.float32),
            jax.ShapeDtypeStruct((_N, 1), jnp.int32),
        ],
        scratch_shapes=[
            pltpu.VMEM((_K, _N), jnp.float32),
        ],
    )(xf, mu, lv, emb_mu)


_NW = 32
_BPW = _N // _NW
_DP = 128


def _sc_lookup(table, idx):
    nc = 2
    mesh = plsc.VectorSubcoreMesh(core_axis_name="c", subcore_axis_name="s")

    @functools.partial(
        pl.kernel,
        mesh=mesh,
        out_type=jax.ShapeDtypeStruct((_N, _DP), jnp.float32),
        scratch_types=[
            pltpu.VMEM((_BPW,), jnp.int32),
            pltpu.VMEM((_BPW, _DP), jnp.float32),
            pltpu.SemaphoreType.DMA,
        ],
    )
    def gather_k(table_hbm, idx_hbm, out_hbm, idx_v, rows_v, sem):
        wid = lax.axis_index("s") * nc + lax.axis_index("c")
        base = wid * _BPW
        pltpu.sync_copy(idx_hbm.at[pl.ds(base, _BPW)], idx_v)
        pltpu.async_copy(table_hbm.at[idx_v], rows_v, sem).wait()
        pltpu.sync_copy(rows_v, out_hbm.at[pl.ds(base, _BPW)])

    padded = jnp.pad(table, ((0, 0), (0, _DP - _D)))
    return gather_k(padded, idx)[:, :_D]


def kernel(x, embeddings_mu, embeddings_logvar, embeddings_pi, batch_mu,
           batch_logvar):
    del embeddings_logvar, embeddings_pi
    b, ch, h, w = x.shape
    xf = jnp.transpose(x, (0, 2, 3, 1)).reshape(-1, _D)
    mu, lv = batch_mu, batch_logvar
    for _ in range(_NUM_ITER):
        mu, lv = _em_iter(xf, mu, lv)
    enc, emb_new, idx = _final_k(xf, mu, lv, embeddings_mu)
    quantized = _sc_lookup(emb_new, idx.reshape(-1))
    qr = jnp.transpose(quantized.reshape(b, h, w, ch), (0, 3, 1, 2))
    return enc, qr


# speedup vs baseline: 9.3508x; 9.3508x over previous
import functools
import math

import jax
import jax.numpy as jnp
from jax import lax
from jax.experimental import pallas as pl
from jax.experimental.pallas import tpu as pltpu
from jax.experimental.pallas import tpu_sc as plsc

_LOG_NORM_CONST = -0.5 * math.log(2 * math.pi)
_NUM_ITER = 3
_BETA = 0.9
_K = 512
_D = 64
_N = 2048


def _tree_last(x):
    n = x.shape[-1]
    while n > 1:
        h = n // 2
        x = x[..., :h] + x[..., h:n]
        n = h
    return x


def _red_k_sum(x):
    acc = x[0:8]
    for i in range(1, 64):
        acc = acc + x[i * 8:(i + 1) * 8]
    n = 8
    while n > 1:
        h = n // 2
        acc = acc[:h] + acc[h:n]
        n = h
    return acc


def _red_n_sum(x):
    acc = x[:, 0:128]
    for i in range(1, 16):
        acc = acc + x[:, i * 128:(i + 1) * 128]
    acc2 = acc[:, 0:8]
    for j in range(1, 16):
        acc2 = acc2 + acc[:, j * 8:(j + 1) * 8]
    return _tree_last(acc2)


def _ll_into(ll_ref, xft_ref, mu_ref, lv_ref):
    xt = xft_ref[:]
    def chunk(c, _):
        base = pl.multiple_of(c * 8, 8)
        mu_c = mu_ref[pl.ds(base, 8)][:, :, None]
        lv_c = lv_ref[pl.ds(base, 8)][:, :, None]
        t = (-0.5 * (lv_c + (xt[None, :, :] - mu_c) ** 2 / jnp.exp(lv_c))
             + _LOG_NORM_CONST)
        acc = None
        for c8 in range(8):
            part = t[:, c8 * 8:(c8 + 1) * 8, :]
            part = part[:, 0:4] + part[:, 4:8]
            part = part[:, 0:2] + part[:, 2:4]
            part = part[:, 0:1] + part[:, 1:2]
            acc = part if acc is None else acc + part
        ll_ref[pl.ds(base, 8)] = acc[:, 0, :]
        return 0
    jax.lax.fori_loop(0, 64, chunk, 0)


_KB = 16


def _em_iter_body(xf_ref, xft_ref, mu_ref, lv_ref, munew_ref, lvnew_ref,
                  ll_ref, p_ref, den_ref):
    _ll_into(ll_ref, xft_ref, mu_ref, lv_ref)
    ll = ll_ref[:]
    m = jnp.max(ll, axis=0, keepdims=True)
    ex = jnp.exp(ll - m)
    s = _red_k_sum(ex)
    p = jnp.exp(ll - (m + jnp.log(s)))
    p_ref[:] = p
    n_k = _red_n_sum(p)
    den_ref[:] = n_k + 1e-6
    pb = p.astype(jnp.bfloat16)
    xb = xf_ref[:].astype(jnp.bfloat16)
    s1 = jax.lax.dot_general(pb, xb, (((1,), (0,)), ((), ())),
                             preferred_element_type=jnp.float32)
    munew_ref[:] = s1 / den_ref[:]

    xfv = xf_ref[:]

    def vblk(i, _):
        base = pl.multiple_of(i * _KB, 8)
        mu_b = munew_ref[pl.ds(base, _KB)]
        a2 = (xfv[None, :, :] - mu_b[:, None, :]) ** 2
        a2b = a2.astype(jnp.bfloat16)
        pb_b = p_ref[pl.ds(base, _KB)].astype(jnp.bfloat16)
        r = jax.lax.dot_general(pb_b[:, None, :], a2b,
                                (((2,), (1,)), ((0,), (0,))),
                                preferred_element_type=jnp.float32)
        var = r[:, 0, :] / den_ref[pl.ds(base, _KB)]
        lvnew_ref[pl.ds(base, _KB)] = jnp.log(jnp.maximum(var, 1e-6))
        return 0
    jax.lax.fori_loop(0, _K // _KB, vblk, 0)


def _em_iter(xf, xft, mu, lv):
    return pl.pallas_call(
        _em_iter_body,
        out_shape=[
            jax.ShapeDtypeStruct((_K, _D), jnp.float32),
            jax.ShapeDtypeStruct((_K, _D), jnp.float32),
        ],
        scratch_shapes=[
            pltpu.VMEM((_K, _N), jnp.float32),
            pltpu.VMEM((_K, _N), jnp.float32),
            pltpu.VMEM((_K, 1), jnp.float32),
        ],
    )(xf, xft, mu, lv)


def _final_body(xft_ref, mu_ref, lv_ref, emb_ref,
                enc_ref, embnew_ref, idx_ref, ll_ref):
    _ll_into(ll_ref, xft_ref, mu_ref, lv_ref)
    lik = jnp.exp(ll_ref[:])
    mx = jnp.max(lik, axis=0, keepdims=True)
    kio = lax.broadcasted_iota(jnp.int32, (_K, _N), 0)
    cand = jnp.where(lik == mx, kio, _K)
    idxr = jnp.min(cand, axis=0, keepdims=True)
    idx_col = jnp.transpose(idxr)
    nio = lax.broadcasted_iota(jnp.int32, (_N, _K), 1)
    enc_ref[:] = (nio == idx_col).astype(jnp.float32)
    idx_ref[:] = idx_col
    embnew_ref[:] = _BETA * emb_ref[:] + (1.0 - _BETA) * mu_ref[:]


def _final_k(xft, mu, lv, emb_mu):
    return pl.pallas_call(
        _final_body,
        out_shape=[
            jax.ShapeDtypeStruct((_N, _K), jnp.float32),
            jax.ShapeDtypeStruct((_K, _D), jnp.float32),
            jax.ShapeDtypeStruct((_N, 1), jnp.int32),
        ],
        scratch_shapes=[
            pltpu.VMEM((_K, _N), jnp.float32),
        ],
    )(xft, mu, lv, emb_mu)


_NW = 32
_BPW = _N // _NW
_DP = 128


def _sc_lookup(table, idx):
    nc = 2
    mesh = plsc.VectorSubcoreMesh(core_axis_name="c", subcore_axis_name="s")

    @functools.partial(
        pl.kernel,
        mesh=mesh,
        out_type=jax.ShapeDtypeStruct((_N, _DP), jnp.float32),
        scratch_types=[
            pltpu.VMEM((_BPW,), jnp.int32),
            pltpu.VMEM((_BPW, _DP), jnp.float32),
            pltpu.SemaphoreType.DMA,
        ],
    )
    def gather_k(table_hbm, idx_hbm, out_hbm, idx_v, rows_v, sem):
        wid = lax.axis_index("s") * nc + lax.axis_index("c")
        base = wid * _BPW
        pltpu.sync_copy(idx_hbm.at[pl.ds(base, _BPW)], idx_v)
        pltpu.async_copy(table_hbm.at[idx_v], rows_v, sem).wait()
        pltpu.sync_copy(rows_v, out_hbm.at[pl.ds(base, _BPW)])

    padded = jnp.pad(table, ((0, 0), (0, _DP - _D)))
    return gather_k(padded, idx)[:, :_D]


def kernel(x, embeddings_mu, embeddings_logvar, embeddings_pi, batch_mu,
           batch_logvar):
    del embeddings_logvar, embeddings_pi
    b, ch, h, w = x.shape
    xf = jnp.transpose(x, (0, 2, 3, 1)).reshape(-1, _D)
    xft = xf.T
    mu, lv = batch_mu, batch_logvar
    for _ in range(_NUM_ITER):
        mu, lv = _em_iter(xf, xft, mu, lv)
    enc, emb_new, idx = _final_k(xft, mu, lv, embeddings_mu)
    quantized = _sc_lookup(emb_new, idx.reshape(-1))
    qr = jnp.transpose(quantized.reshape(b, h, w, ch), (0, 3, 1, 2))
    return enc, qr


# speedup vs baseline: 9.6372x; 1.0306x over previous
import functools
import math

import jax
import jax.numpy as jnp
from jax import lax
from jax.experimental import pallas as pl
from jax.experimental.pallas import tpu as pltpu
from jax.experimental.pallas import tpu_sc as plsc

_LOG_NORM_CONST = -0.5 * math.log(2 * math.pi)
_NUM_ITER = 3
_BETA = 0.9
_K = 512
_D = 64
_N = 2048


def _tree_last(x):
    n = x.shape[-1]
    while n > 1:
        h = n // 2
        x = x[..., :h] + x[..., h:n]
        n = h
    return x


def _red_k_sum(x):
    acc = x[0:8]
    for i in range(1, 64):
        acc = acc + x[i * 8:(i + 1) * 8]
    n = 8
    while n > 1:
        h = n // 2
        acc = acc[:h] + acc[h:n]
        n = h
    return acc


def _red_n_sum(x):
    acc = x[:, 0:128]
    for i in range(1, 16):
        acc = acc + x[:, i * 128:(i + 1) * 128]
    acc2 = acc[:, 0:8]
    for j in range(1, 16):
        acc2 = acc2 + acc[:, j * 8:(j + 1) * 8]
    return _tree_last(acc2)


def _ll_into(ll_ref, xft_ref, mu_ref, lv_ref):
    xt = xft_ref[:]
    def chunk(c, _):
        base = pl.multiple_of(c * 16, 8)
        mu_c = mu_ref[pl.ds(base, 16)][:, :, None]
        lv_c = lv_ref[pl.ds(base, 16)][:, :, None]
        t = (-0.5 * (lv_c + (xt[None, :, :] - mu_c) ** 2 / jnp.exp(lv_c))
             + _LOG_NORM_CONST)
        acc = None
        for c8 in range(8):
            part = t[:, c8 * 8:(c8 + 1) * 8, :]
            part = part[:, 0:4] + part[:, 4:8]
            part = part[:, 0:2] + part[:, 2:4]
            part = part[:, 0:1] + part[:, 1:2]
            acc = part if acc is None else acc + part
        ll_ref[pl.ds(base, 16)] = acc[:, 0, :]
        return 0
    jax.lax.fori_loop(0, 32, chunk, 0)


_KB = 32


def _em_iter_body(xf_ref, xft_ref, mu_ref, lv_ref, munew_ref, lvnew_ref,
                  ll_ref, p_ref, den_ref):
    _ll_into(ll_ref, xft_ref, mu_ref, lv_ref)
    ll = ll_ref[:]
    m = jnp.max(ll, axis=0, keepdims=True)
    ex = jnp.exp(ll - m)
    s = _red_k_sum(ex)
    p = jnp.exp(ll - (m + jnp.log(s)))
    p_ref[:] = p
    n_k = _red_n_sum(p)
    den_ref[:] = n_k + 1e-6
    pb = p.astype(jnp.bfloat16)
    xb = xf_ref[:].astype(jnp.bfloat16)
    s1 = jax.lax.dot_general(pb, xb, (((1,), (0,)), ((), ())),
                             preferred_element_type=jnp.float32)
    munew_ref[:] = s1 / den_ref[:]

    xfv = xf_ref[:]

    def vblk(i, _):
        base = pl.multiple_of(i * _KB, 8)
        mu_b = munew_ref[pl.ds(base, _KB)]
        a2 = (xfv[None, :, :] - mu_b[:, None, :]) ** 2
        a2b = a2.astype(jnp.bfloat16)
        pb_b = p_ref[pl.ds(base, _KB)].astype(jnp.bfloat16)
        r = jax.lax.dot_general(pb_b[:, None, :], a2b,
                                (((2,), (1,)), ((0,), (0,))),
                                preferred_element_type=jnp.float32)
        var = r[:, 0, :] / den_ref[pl.ds(base, _KB)]
        lvnew_ref[pl.ds(base, _KB)] = jnp.log(jnp.maximum(var, 1e-6))
        return 0
    jax.lax.fori_loop(0, _K // _KB, vblk, 0)


def _em_iter(xf, xft, mu, lv):
    return pl.pallas_call(
        _em_iter_body,
        out_shape=[
            jax.ShapeDtypeStruct((_K, _D), jnp.float32),
            jax.ShapeDtypeStruct((_K, _D), jnp.float32),
        ],
        scratch_shapes=[
            pltpu.VMEM((_K, _N), jnp.float32),
            pltpu.VMEM((_K, _N), jnp.float32),
            pltpu.VMEM((_K, 1), jnp.float32),
        ],
    )(xf, xft, mu, lv)


def _final_body(xft_ref, mu_ref, lv_ref, emb_ref,
                enc_ref, embnew_ref, idx_ref, ll_ref):
    _ll_into(ll_ref, xft_ref, mu_ref, lv_ref)
    lik = jnp.exp(ll_ref[:])
    mx = jnp.max(lik, axis=0, keepdims=True)
    kio = lax.broadcasted_iota(jnp.int32, (_K, _N), 0)
    cand = jnp.where(lik == mx, kio, _K)
    idxr = jnp.min(cand, axis=0, keepdims=True)
    idx_col = jnp.transpose(idxr)
    nio = lax.broadcasted_iota(jnp.int32, (_N, _K), 1)
    enc_ref[:] = (nio == idx_col).astype(jnp.float32)
    idx_ref[:] = idx_col
    embnew_ref[:] = _BETA * emb_ref[:] + (1.0 - _BETA) * mu_ref[:]


def _final_k(xft, mu, lv, emb_mu):
    return pl.pallas_call(
        _final_body,
        out_shape=[
            jax.ShapeDtypeStruct((_N, _K), jnp.float32),
            jax.ShapeDtypeStruct((_K, _D), jnp.float32),
            jax.ShapeDtypeStruct((_N, 1), jnp.int32),
        ],
        scratch_shapes=[
            pltpu.VMEM((_K, _N), jnp.float32),
        ],
    )(xft, mu, lv, emb_mu)


_NW = 32
_BPW = _N // _NW
_DP = 128


def _sc_lookup(table, idx):
    nc = 2
    mesh = plsc.VectorSubcoreMesh(core_axis_name="c", subcore_axis_name="s")

    @functools.partial(
        pl.kernel,
        mesh=mesh,
        out_type=jax.ShapeDtypeStruct((_N, _DP), jnp.float32),
        scratch_types=[
            pltpu.VMEM((_BPW,), jnp.int32),
            pltpu.VMEM((_BPW, _DP), jnp.float32),
            pltpu.SemaphoreType.DMA,
        ],
    )
    def gather_k(table_hbm, idx_hbm, out_hbm, idx_v, rows_v, sem):
        wid = lax.axis_index("s") * nc + lax.axis_index("c")
        base = wid * _BPW
        pltpu.sync_copy(idx_hbm.at[pl.ds(base, _BPW)], idx_v)
        pltpu.async_copy(table_hbm.at[idx_v], rows_v, sem).wait()
        pltpu.sync_copy(rows_v, out_hbm.at[pl.ds(base, _BPW)])

    padded = jnp.pad(table, ((0, 0), (0, _DP - _D)))
    return gather_k(padded, idx)[:, :_D]


def kernel(x, embeddings_mu, embeddings_logvar, embeddings_pi, batch_mu,
           batch_logvar):
    del embeddings_logvar, embeddings_pi
    b, ch, h, w = x.shape
    xf = jnp.transpose(x, (0, 2, 3, 1)).reshape(-1, _D)
    xft = xf.T
    mu, lv = batch_mu, batch_logvar
    for _ in range(_NUM_ITER):
        mu, lv = _em_iter(xf, xft, mu, lv)
    enc, emb_new, idx = _final_k(xft, mu, lv, embeddings_mu)
    quantized = _sc_lookup(emb_new, idx.reshape(-1))
    qr = jnp.transpose(quantized.reshape(b, h, w, ch), (0, 3, 1, 2))
    return enc, qr


# speedup vs baseline: 10.1299x; 1.0511x over previous
import functools
import math

import jax
import jax.numpy as jnp
from jax import lax
from jax.experimental import pallas as pl
from jax.experimental.pallas import tpu as pltpu
from jax.experimental.pallas import tpu_sc as plsc

_LOG_NORM_CONST = -0.5 * math.log(2 * math.pi)
_NUM_ITER = 3
_BETA = 0.9
_K = 512
_D = 64
_N = 2048


def _tree_last(x):
    n = x.shape[-1]
    while n > 1:
        h = n // 2
        x = x[..., :h] + x[..., h:n]
        n = h
    return x


def _red_k_sum(x):
    acc = x[0:8]
    for i in range(1, 64):
        acc = acc + x[i * 8:(i + 1) * 8]
    n = 8
    while n > 1:
        h = n // 2
        acc = acc[:h] + acc[h:n]
        n = h
    return acc


def _red_n_sum(x):
    acc = x[:, 0:128]
    for i in range(1, 16):
        acc = acc + x[:, i * 128:(i + 1) * 128]
    acc2 = acc[:, 0:8]
    for j in range(1, 16):
        acc2 = acc2 + acc[:, j * 8:(j + 1) * 8]
    return _tree_last(acc2)


def _ll_into(ll_ref, xft_ref, mu_ref, lv_ref):
    xt = xft_ref[:]
    def chunk(c, _):
        base = pl.multiple_of(c * 16, 8)
        mu_c = mu_ref[pl.ds(base, 16)][:, :, None]
        lv_c = lv_ref[pl.ds(base, 16)][:, :, None]
        t = (-0.5 * (lv_c + (xt[None, :, :] - mu_c) ** 2 / jnp.exp(lv_c))
             + _LOG_NORM_CONST)
        acc = None
        for c8 in range(8):
            part = t[:, c8 * 8:(c8 + 1) * 8, :]
            part = part[:, 0:4] + part[:, 4:8]
            part = part[:, 0:2] + part[:, 2:4]
            part = part[:, 0:1] + part[:, 1:2]
            acc = part if acc is None else acc + part
        ll_ref[pl.ds(base, 16)] = acc[:, 0, :]
        return 0
    jax.lax.fori_loop(0, 32, chunk, 0)


_KB = 32


def _em_iter_body(xf_ref, xft_ref, mu_ref, lv_ref, munew_ref, lvnew_ref,
                  ll_ref, p_ref, den_ref):
    _ll_into(ll_ref, xft_ref, mu_ref, lv_ref)
    ll = ll_ref[:]
    m = jnp.max(ll, axis=0, keepdims=True)
    ex = jnp.exp(ll - m)
    s = _red_k_sum(ex)
    p = jnp.exp(ll - (m + jnp.log(s)))
    p_ref[:] = p
    n_k = _red_n_sum(p)
    den_ref[:] = n_k + 1e-6
    pb = p.astype(jnp.bfloat16)
    xb = xf_ref[:].astype(jnp.bfloat16)
    s1 = jax.lax.dot_general(pb, xb, (((1,), (0,)), ((), ())),
                             preferred_element_type=jnp.float32)
    munew_ref[:] = s1 / den_ref[:]

    xtv = xft_ref[:]

    def vblk(i, _):
        base = pl.multiple_of(i * _KB, 8)
        mu_b = munew_ref[pl.ds(base, _KB)][:, :, None]
        a2 = (xtv[None, :, :] - mu_b) ** 2
        a2b = a2.astype(jnp.bfloat16)
        pb_b = p_ref[pl.ds(base, _KB)].astype(jnp.bfloat16)
        r = jax.lax.dot_general(pb_b[:, None, :], a2b,
                                (((2,), (2,)), ((0,), (0,))),
                                preferred_element_type=jnp.float32)
        var = r[:, 0, :] / den_ref[pl.ds(base, _KB)]
        lvnew_ref[pl.ds(base, _KB)] = jnp.log(jnp.maximum(var, 1e-6))
        return 0
    jax.lax.fori_loop(0, _K // _KB, vblk, 0)


def _em_iter(xf, xft, mu, lv):
    return pl.pallas_call(
        _em_iter_body,
        out_shape=[
            jax.ShapeDtypeStruct((_K, _D), jnp.float32),
            jax.ShapeDtypeStruct((_K, _D), jnp.float32),
        ],
        scratch_shapes=[
            pltpu.VMEM((_K, _N), jnp.float32),
            pltpu.VMEM((_K, _N), jnp.float32),
            pltpu.VMEM((_K, 1), jnp.float32),
        ],
    )(xf, xft, mu, lv)


def _final_body(xft_ref, mu_ref, lv_ref, emb_ref,
                enc_ref, embnew_ref, idx_ref, ll_ref):
    _ll_into(ll_ref, xft_ref, mu_ref, lv_ref)
    lik = jnp.exp(ll_ref[:])
    mx = jnp.max(lik, axis=0, keepdims=True)
    kio = lax.broadcasted_iota(jnp.int32, (_K, _N), 0)
    cand = jnp.where(lik == mx, kio, _K)
    idxr = jnp.min(cand, axis=0, keepdims=True)
    idx_col = jnp.transpose(idxr)
    nio = lax.broadcasted_iota(jnp.int32, (_N, _K), 1)
    enc_ref[:] = (nio == idx_col).astype(jnp.float32)
    idx_ref[:] = idx_col
    embnew_ref[:] = _BETA * emb_ref[:] + (1.0 - _BETA) * mu_ref[:]


def _final_k(xft, mu, lv, emb_mu):
    return pl.pallas_call(
        _final_body,
        out_shape=[
            jax.ShapeDtypeStruct((_N, _K), jnp.float32),
            jax.ShapeDtypeStruct((_K, _D), jnp.float32),
            jax.ShapeDtypeStruct((_N, 1), jnp.int32),
        ],
        scratch_shapes=[
            pltpu.VMEM((_K, _N), jnp.float32),
        ],
    )(xft, mu, lv, emb_mu)


_NW = 32
_BPW = _N // _NW
_DP = 128


def _sc_lookup(table, idx):
    nc = 2
    mesh = plsc.VectorSubcoreMesh(core_axis_name="c", subcore_axis_name="s")

    @functools.partial(
        pl.kernel,
        mesh=mesh,
        out_type=jax.ShapeDtypeStruct((_N, _DP), jnp.float32),
        scratch_types=[
            pltpu.VMEM((_BPW,), jnp.int32),
            pltpu.VMEM((_BPW, _DP), jnp.float32),
            pltpu.SemaphoreType.DMA,
        ],
    )
    def gather_k(table_hbm, idx_hbm, out_hbm, idx_v, rows_v, sem):
        wid = lax.axis_index("s") * nc + lax.axis_index("c")
        base = wid * _BPW
        pltpu.sync_copy(idx_hbm.at[pl.ds(base, _BPW)], idx_v)
        pltpu.async_copy(table_hbm.at[idx_v], rows_v, sem).wait()
        pltpu.sync_copy(rows_v, out_hbm.at[pl.ds(base, _BPW)])

    padded = jnp.pad(table, ((0, 0), (0, _DP - _D)))
    return gather_k(padded, idx)[:, :_D]


def kernel(x, embeddings_mu, embeddings_logvar, embeddings_pi, batch_mu,
           batch_logvar):
    del embeddings_logvar, embeddings_pi
    b, ch, h, w = x.shape
    xf = jnp.transpose(x, (0, 2, 3, 1)).reshape(-1, _D)
    xft = xf.T
    mu, lv = batch_mu, batch_logvar
    for _ in range(_NUM_ITER):
        mu, lv = _em_iter(xf, xft, mu, lv)
    enc, emb_new, idx = _final_k(xft, mu, lv, embeddings_mu)
    quantized = _sc_lookup(emb_new, idx.reshape(-1))
    qr = jnp.transpose(quantized.reshape(b, h, w, ch), (0, 3, 1, 2))
    return enc, qr
